# baseline (speedup 1.0000x reference)
import jax
import jax.numpy as jnp
from jax import lax
from jax.experimental import pallas as pl
from jax.experimental.pallas import tpu as pltpu
from jax.experimental.pallas import tpu_sc as plsc

N = 10000
E = 320000
F_IN = 128
HID = 16
HEADS = 8
HF = HEADS * HID
OUT = 2

NC = 2
NS = 16
NW = NC * NS
EPW = E // NW
CHUNK = 80
NCH = EPW // CHUNK
ZR = 624
ZREM = N - NS * ZR
ZCH = 48

ROW_BLK = 1000
GRID = N // ROW_BLK


def _lrelu(x):
    return jnp.where(x >= 0, x, 0.2 * x)


def _stage_a_body(x_ref, w1_ref, as_ref, ad_ref, h1_ref, s1_ref, d1_ref):
    h = jnp.dot(x_ref[...], w1_ref[...], preferred_element_type=jnp.float32)
    h1_ref[...] = h
    z = jnp.zeros((ROW_BLK, 8), jnp.float32)
    s1_ref[...] = jnp.concatenate(
        [jnp.dot(h, as_ref[...], preferred_element_type=jnp.float32), z], axis=1)
    d1_ref[...] = jnp.concatenate(
        [jnp.dot(h, ad_ref[...], preferred_element_type=jnp.float32), z], axis=1)


def _stage_a(x, w1, a_s, a_d):
    return pl.pallas_call(
        _stage_a_body,
        grid=(GRID,),
        in_specs=[
            pl.BlockSpec((ROW_BLK, F_IN), lambda i: (i, 0)),
            pl.BlockSpec((F_IN, HF), lambda i: (0, 0)),
            pl.BlockSpec((HF, HEADS), lambda i: (0, 0)),
            pl.BlockSpec((HF, HEADS), lambda i: (0, 0)),
        ],
        out_specs=[
            pl.BlockSpec((ROW_BLK, HF), lambda i: (i, 0)),
            pl.BlockSpec((ROW_BLK, 16), lambda i: (i, 0)),
            pl.BlockSpec((ROW_BLK, 16), lambda i: (i, 0)),
        ],
        out_shape=[
            jax.ShapeDtypeStruct((N, HF), jnp.float32),
            jax.ShapeDtypeStruct((N, 16), jnp.float32),
            jax.ShapeDtypeStruct((N, 16), jnp.float32),
        ],
    )(x, w1, a_s, a_d)


def _edge1_body(h1_hbm, s1_hbm, d1_hbm, src_hbm, dst_hbm,
                p1h_hbm, p1w_hbm,
                acc_h, acc_w,
                srcv0, srcv1, dstv0, dstv1, tbuf0, tbuf1,
                sbuf0, sbuf1, dbuf0, dbuf1, wbuf0, wbuf1,
                gh0, gs0, gd0, gh1, gs1, gd1, sh0, sw0, sh1, sw1):
    cid = lax.axis_index("c")
    sid = lax.axis_index("s")
    wid = sid * NC + cid
    row0 = sid * ZR
    srcv = (srcv0, srcv1)
    dstv = (dstv0, dstv1)
    tbuf = (tbuf0, tbuf1)
    sbuf = (sbuf0, sbuf1)
    dbuf = (dbuf0, dbuf1)
    wbuf = (wbuf0, wbuf1)
    gsem = ((gh0, gs0, gd0), (gh1, gs1, gd1))
    ssem = ((sh0, sw0), (sh1, sw1))

    @pl.loop(0, CHUNK)
    def _(b):
        for j in range(HF // 16):
            tbuf0[b, pl.ds(16 * j, 16)] = jnp.zeros((16,), jnp.float32)
        wbuf0[b, :] = jnp.zeros((16,), jnp.float32)

    for j in range(ZR // CHUNK):
        pltpu.sync_copy(tbuf0.at[pl.ds(0, CHUNK)],
                        acc_h.at[pl.ds(row0 + j * CHUNK, CHUNK)])
        pltpu.sync_copy(wbuf0.at[pl.ds(0, CHUNK)],
                        acc_w.at[pl.ds(row0 + j * CHUNK, CHUNK)])
    ztail = ZR - (ZR // CHUNK) * CHUNK
    pltpu.sync_copy(tbuf0.at[pl.ds(0, ztail)],
                    acc_h.at[pl.ds(row0 + ZR - ztail, ztail)])
    pltpu.sync_copy(wbuf0.at[pl.ds(0, ztail)],
                    acc_w.at[pl.ds(row0 + ZR - ztail, ztail)])

    @pl.when(sid == 0)
    def _():
        pltpu.sync_copy(tbuf0.at[pl.ds(0, ZREM)], acc_h.at[pl.ds(NS * ZR, ZREM)])
        pltpu.sync_copy(wbuf0.at[pl.ds(0, ZREM)], acc_w.at[pl.ds(NS * ZR, ZREM)])
    plsc.subcore_barrier()

    base = wid * EPW

    def issue(g, p):
        off = base + g * CHUNK
        pltpu.sync_copy(src_hbm.at[pl.ds(off, CHUNK)], srcv[p])
        pltpu.sync_copy(dst_hbm.at[pl.ds(off, CHUNK)], dstv[p])
        pltpu.async_copy(h1_hbm.at[srcv[p]], tbuf[p], gsem[p][0])
        pltpu.async_copy(s1_hbm.at[srcv[p]], sbuf[p], gsem[p][1])
        pltpu.async_copy(d1_hbm.at[dstv[p]], dbuf[p], gsem[p][2])

    def gwait(p):
        pltpu.make_async_copy(h1_hbm.at[srcv[p]], tbuf[p], gsem[p][0]).wait()
        pltpu.make_async_copy(s1_hbm.at[srcv[p]], sbuf[p], gsem[p][1]).wait()
        pltpu.make_async_copy(d1_hbm.at[dstv[p]], dbuf[p], gsem[p][2]).wait()

    def sissue(p):
        pltpu.async_copy(tbuf[p], acc_h.at[dstv[p]], ssem[p][0], add=True)
        pltpu.async_copy(wbuf[p], acc_w.at[dstv[p]], ssem[p][1], add=True)

    def swait(p):
        pltpu.make_async_copy(tbuf[p], acc_h.at[dstv[p]], ssem[p][0]).wait()
        pltpu.make_async_copy(wbuf[p], acc_w.at[dstv[p]], ssem[p][1]).wait()

    def compute(p):
        tb = tbuf[p]
        sb = sbuf[p]
        db = dbuf[p]
        wb = wbuf[p]

        @pl.loop(0, CHUNK)
        def _(b):
            al = sb[b, :] + db[b, :]
            w16 = jnp.exp(_lrelu(al))
            wb[b, :] = w16
            for hh in range(HEADS):
                tb[b, pl.ds(16 * hh, 16)] = tb[b, pl.ds(16 * hh, 16)] * w16[hh]

    issue(0, 0)

    @pl.loop(0, NCH // 2)
    def _(k):
        g0 = 2 * k
        gwait(0)

        @pl.when(g0 >= 1)
        def _():
            swait(1)
        issue(g0 + 1, 1)
        compute(0)
        sissue(0)
        gwait(1)
        swait(0)

        @pl.when(g0 + 2 < NCH)
        def _():
            issue(g0 + 2, 0)
        compute(1)
        sissue(1)

    gwait(0)
    swait(1)
    compute(0)
    sissue(0)
    swait(0)

    plsc.subcore_barrier()
    pltpu.sync_copy(acc_h.at[pl.ds(row0, ZR)], p1h_hbm.at[cid, pl.ds(row0, ZR)])
    pltpu.sync_copy(acc_w.at[pl.ds(row0, ZR)], p1w_hbm.at[cid, pl.ds(row0, ZR)])

    @pl.when(sid == 0)
    def _():
        pltpu.sync_copy(acc_h.at[pl.ds(NS * ZR, ZREM)],
                        p1h_hbm.at[cid, pl.ds(NS * ZR, ZREM)])
        pltpu.sync_copy(acc_w.at[pl.ds(NS * ZR, ZREM)],
                        p1w_hbm.at[cid, pl.ds(NS * ZR, ZREM)])


def _edge1(h1, s1, d1, src, dst):
    mesh = plsc.VectorSubcoreMesh(core_axis_name="c", subcore_axis_name="s")
    return pl.kernel(
        _edge1_body,
        out_type=[
            jax.ShapeDtypeStruct((NC, N, HF), jnp.float32),
            jax.ShapeDtypeStruct((NC, N, 16), jnp.float32),
        ],
        mesh=mesh,
        compiler_params=pltpu.CompilerParams(use_tc_tiling_on_sc=False,
                                             needs_layout_passes=False),
        scratch_types=(
            [pltpu.VMEM_SHARED((N, HF), jnp.float32),
             pltpu.VMEM_SHARED((N, 16), jnp.float32)]
            + [pltpu.VMEM((CHUNK,), jnp.int32)] * 4
            + [pltpu.VMEM((CHUNK, HF), jnp.float32)] * 2
            + [pltpu.VMEM((CHUNK, 16), jnp.float32)] * 6
            + [pltpu.SemaphoreType.DMA] * 10
        ),
    )(h1, s1, d1, src, dst)


def _stage_c_body(p1h0_ref, p1h1_ref, p1w0_ref, p1w1_ref, h1_ref, s1_ref,
                  d1_ref, b1_ref, w2_ref, as2_ref, ad2_ref, t2_ref):
    asrc = s1_ref[...][:, :HEADS]
    adst = d1_ref[...][:, :HEADS]
    wself = jnp.exp(_lrelu(asrc + adst))
    wself_x = jnp.broadcast_to(wself[:, :, None], (ROW_BLK, HEADS, HID))
    wself_x = wself_x.reshape(ROW_BLK, HF)
    h1 = h1_ref[...]
    s = p1h0_ref[0] + p1h1_ref[0] + wself_x * h1
    den = p1w0_ref[0][:, :HEADS] + p1w1_ref[0][:, :HEADS] + wself
    den_x = jnp.broadcast_to(den[:, :, None], (ROW_BLK, HEADS, HID))
    den_x = den_x.reshape(ROW_BLK, HF)
    o1 = s / (den_x + 1e-16) + b1_ref[...][0]
    e1 = jnp.where(o1 > 0, o1, jnp.exp(o1) - 1.0)
    h2 = jnp.dot(e1, w2_ref[...], preferred_element_type=jnp.float32)
    a20 = as2_ref[0, 0]
    a21 = as2_ref[0, 1]
    b20 = ad2_ref[0, 0]
    b21 = ad2_ref[0, 1]
    asrc2 = h2[:, 0] * a20 + h2[:, 1] * a21
    adst2 = h2[:, 0] * b20 + h2[:, 1] * b21
    t2_ref[...] = jnp.stack([h2[:, 0], h2[:, 1], asrc2, adst2], axis=1)


def _stage_c(p1h, p1w, h1, s1, d1, b1, w2, as2, ad2):
    return pl.pallas_call(
        _stage_c_body,
        grid=(GRID,),
        in_specs=[
            pl.BlockSpec((1, ROW_BLK, HF), lambda i: (0, i, 0)),
            pl.BlockSpec((1, ROW_BLK, HF), lambda i: (1, i, 0)),
            pl.BlockSpec((1, ROW_BLK, 16), lambda i: (0, i, 0)),
            pl.BlockSpec((1, ROW_BLK, 16), lambda i: (1, i, 0)),
            pl.BlockSpec((ROW_BLK, HF), lambda i: (i, 0)),
            pl.BlockSpec((ROW_BLK, 16), lambda i: (i, 0)),
            pl.BlockSpec((ROW_BLK, 16), lambda i: (i, 0)),
            pl.BlockSpec((1, HF), lambda i: (0, 0)),
            pl.BlockSpec((HF, OUT), lambda i: (0, 0)),
            pl.BlockSpec((1, OUT), lambda i: (0, 0)),
            pl.BlockSpec((1, OUT), lambda i: (0, 0)),
        ],
        out_specs=pl.BlockSpec((ROW_BLK, 4), lambda i: (i, 0)),
        out_shape=jax.ShapeDtypeStruct((N, 4), jnp.float32),
    )(p1h, p1h, p1w, p1w, h1, s1, d1, b1, w2, as2, ad2)


def _edge2_body(t2_hbm, src_hbm, dst_hbm, p2_hbm,
                acc2, t2v, srcv, dstv, mbuf, zbuf):
    cid = lax.axis_index("c")
    sid = lax.axis_index("s")
    wid = sid * NC + cid
    row0 = sid * ZR

    @pl.loop(0, ZCH)
    def _(b):
        zbuf[b, :] = jnp.zeros((16,), jnp.float32)

    for j in range(ZR // ZCH):
        pltpu.sync_copy(zbuf.at[pl.ds(0, ZCH)], acc2.at[pl.ds(row0 + j * ZCH, ZCH)])

    @pl.when(sid == 0)
    def _():
        pltpu.sync_copy(zbuf.at[pl.ds(0, ZREM)], acc2.at[pl.ds(NS * ZR, ZREM)])
    pltpu.sync_copy(t2_hbm, t2v)
    plsc.subcore_barrier()

    base = wid * EPW
    lane = jnp.arange(16, dtype=jnp.int32)

    @pl.loop(0, NCH)
    def _(g):
        off = base + g * CHUNK
        pltpu.sync_copy(src_hbm.at[pl.ds(off, CHUNK)], srcv)
        pltpu.sync_copy(dst_hbm.at[pl.ds(off, CHUNK)], dstv)
        for j in range(CHUNK // 16):
            s16 = srcv[pl.ds(j * 16, 16)]
            d16 = dstv[pl.ds(j * 16, 16)]
            sa = plsc.load_gather(t2v, [s16, jnp.full((16,), 2, jnp.int32)])
            da = plsc.load_gather(t2v, [d16, jnp.full((16,), 3, jnp.int32)])
            w = jnp.exp(_lrelu(sa + da))
            m0 = plsc.load_gather(t2v, [s16, jnp.full((16,), 0, jnp.int32)]) * w
            m1 = plsc.load_gather(t2v, [s16, jnp.full((16,), 1, jnp.int32)]) * w
            rows = j * 16 + lane
            plsc.store_scatter(mbuf, [rows, jnp.full((16,), 0, jnp.int32)], m0)
            plsc.store_scatter(mbuf, [rows, jnp.full((16,), 1, jnp.int32)], m1)
            plsc.store_scatter(mbuf, [rows, jnp.full((16,), 2, jnp.int32)], w)
        pltpu.sync_copy(mbuf, acc2.at[dstv], add=True)

    plsc.subcore_barrier()
    pltpu.sync_copy(acc2.at[pl.ds(row0, ZR)], p2_hbm.at[cid, pl.ds(row0, ZR)])

    @pl.when(sid == 0)
    def _():
        pltpu.sync_copy(acc2.at[pl.ds(NS * ZR, ZREM)],
                        p2_hbm.at[cid, pl.ds(NS * ZR, ZREM)])


def _edge2(t2, src, dst):
    mesh = plsc.VectorSubcoreMesh(core_axis_name="c", subcore_axis_name="s")
    return pl.kernel(
        _edge2_body,
        out_type=jax.ShapeDtypeStruct((NC, N, 16), jnp.float32),
        mesh=mesh,
        compiler_params=pltpu.CompilerParams(use_tc_tiling_on_sc=False,
                                             needs_layout_passes=False),
        scratch_types=[
            pltpu.VMEM_SHARED((N, 16), jnp.float32),
            pltpu.VMEM((N, 4), jnp.float32),
            pltpu.VMEM((CHUNK,), jnp.int32),
            pltpu.VMEM((CHUNK,), jnp.int32),
            pltpu.VMEM((CHUNK, 16), jnp.float32),
            pltpu.VMEM((ZCH, 16), jnp.float32),
        ],
    )(t2, src, dst)


def _stage_e_body(p20_ref, p21_ref, t2_ref, b2_ref, out_ref):
    t2 = t2_ref[...]
    wself = jnp.exp(_lrelu(t2[:, 2] + t2[:, 3]))
    s0 = p20_ref[0][:, 0] + p21_ref[0][:, 0] + wself * t2[:, 0]
    s1 = p20_ref[0][:, 1] + p21_ref[0][:, 1] + wself * t2[:, 1]
    den = p20_ref[0][:, 2] + p21_ref[0][:, 2] + wself + 1e-16
    out_ref[...] = (jnp.stack([s0, s1], axis=1) / den[:, None]
                    + b2_ref[...][0])


def _stage_e(p2, t2, b2):
    return pl.pallas_call(
        _stage_e_body,
        grid=(GRID,),
        in_specs=[
            pl.BlockSpec((1, ROW_BLK, 16), lambda i: (0, i, 0)),
            pl.BlockSpec((1, ROW_BLK, 16), lambda i: (1, i, 0)),
            pl.BlockSpec((ROW_BLK, 4), lambda i: (i, 0)),
            pl.BlockSpec((1, OUT), lambda i: (0, 0)),
        ],
        out_specs=pl.BlockSpec((ROW_BLK, OUT), lambda i: (i, 0)),
        out_shape=jax.ShapeDtypeStruct((N, OUT), jnp.float32),
    )(p2, p2, t2, b2)


def kernel(x, edge_index, W1, att_src1, att_dst1, b1, W2, att_src2, att_dst2, b2):
    src = edge_index[0]
    dst = edge_index[1]
    eye = jnp.eye(HEADS, dtype=jnp.float32)
    a_s = (att_src1[:, :, None] * eye[:, None, :]).reshape(HF, HEADS)
    a_d = (att_dst1[:, :, None] * eye[:, None, :]).reshape(HF, HEADS)

    h1, s1, d1 = _stage_a(x, W1, a_s, a_d)
    p1h, p1w = _edge1(h1, s1, d1, src, dst)
    t2 = _stage_c(p1h, p1w, h1, s1, d1, b1.reshape(1, HF), W2,
                  att_src2, att_dst2)
    p2 = _edge2(t2, src, dst)
    return _stage_e(p2, t2, b2.reshape(1, OUT))


# speedup vs baseline: 1.4165x; 1.4165x over previous
import jax
import jax.numpy as jnp
from jax import lax
from jax.experimental import pallas as pl
from jax.experimental.pallas import tpu as pltpu
from jax.experimental.pallas import tpu_sc as plsc

N = 10000
E = 320000
F_IN = 128
HID = 16
HEADS = 8
HF = HEADS * HID
TW = HF + 16
OUT = 2

NC = 2
NS = 16
NW = NC * NS
EPW = E // NW
CHUNK = 80
NSEC = 5
SECE = EPW // NSEC
SECC = SECE // CHUNK
ZR = 624
ZREM = N - NS * ZR

ROW_BLK = 1000
GRID = N // ROW_BLK


def _lrelu(x):
    return jnp.where(x >= 0, x, 0.2 * x)


def _stage_a_body(x_ref, w1_ref, as_ref, ad_ref, t1_ref, d1_ref):
    h = jnp.dot(x_ref[...], w1_ref[...], preferred_element_type=jnp.float32)
    z = jnp.zeros((ROW_BLK, 8), jnp.float32)
    t1_ref[...] = jnp.concatenate(
        [h, jnp.dot(h, as_ref[...], preferred_element_type=jnp.float32), z],
        axis=1)
    d1_ref[...] = jnp.concatenate(
        [jnp.dot(h, ad_ref[...], preferred_element_type=jnp.float32), z], axis=1)


def _stage_a(x, w1, a_s, a_d):
    return pl.pallas_call(
        _stage_a_body,
        grid=(GRID,),
        in_specs=[
            pl.BlockSpec((ROW_BLK, F_IN), lambda i: (i, 0)),
            pl.BlockSpec((F_IN, HF), lambda i: (0, 0)),
            pl.BlockSpec((HF, HEADS), lambda i: (0, 0)),
            pl.BlockSpec((HF, HEADS), lambda i: (0, 0)),
        ],
        out_specs=[
            pl.BlockSpec((ROW_BLK, TW), lambda i: (i, 0)),
            pl.BlockSpec((ROW_BLK, 16), lambda i: (i, 0)),
        ],
        out_shape=[
            jax.ShapeDtypeStruct((N, TW), jnp.float32),
            jax.ShapeDtypeStruct((N, 16), jnp.float32),
        ],
    )(x, w1, a_s, a_d)


def _edge1_body(t1_hbm, d1_hbm, src_hbm, dst_hbm, p1_hbm,
                acc, srcsec, dstsec, dstv0, dstv1, tbuf0, tbuf1,
                dbuf0, dbuf1,
                gt0, gd0, gt1, gd1, ss0, ss1):
    cid = lax.axis_index("c")
    sid = lax.axis_index("s")
    wid = sid * NC + cid
    row0 = sid * ZR
    dstv = (dstv0, dstv1)
    tbuf = (tbuf0, tbuf1)
    dbuf = (dbuf0, dbuf1)
    gsem = ((gt0, gd0), (gt1, gd1))
    ssem = (ss0, ss1)

    @pl.loop(0, CHUNK)
    def _(b):
        for j in range(TW // 16):
            tbuf0[b, pl.ds(16 * j, 16)] = jnp.zeros((16,), jnp.float32)

    for j in range(ZR // CHUNK):
        pltpu.sync_copy(tbuf0.at[pl.ds(0, CHUNK)],
                        acc.at[pl.ds(row0 + j * CHUNK, CHUNK)])
    ztail = ZR - (ZR // CHUNK) * CHUNK
    pltpu.sync_copy(tbuf0.at[pl.ds(0, ztail)],
                    acc.at[pl.ds(row0 + ZR - ztail, ztail)])

    @pl.when(sid == 0)
    def _():
        pltpu.sync_copy(tbuf0.at[pl.ds(0, ZREM)], acc.at[pl.ds(NS * ZR, ZREM)])
    plsc.subcore_barrier()

    base = wid * EPW

    def issue(l, p):
        sl = pl.ds(l * CHUNK, CHUNK)
        for i in range(CHUNK // 16):
            dstv[p][pl.ds(16 * i, 16)] = dstsec[pl.ds(l * CHUNK + 16 * i, 16)]
        pltpu.async_copy(t1_hbm.at[srcsec.at[sl]], tbuf[p], gsem[p][0])
        pltpu.async_copy(d1_hbm.at[dstsec.at[sl]], dbuf[p], gsem[p][1])

    def gwait(l, p):
        sl = pl.ds(l * CHUNK, CHUNK)
        pltpu.make_async_copy(t1_hbm.at[srcsec.at[sl]], tbuf[p], gsem[p][0]).wait()
        pltpu.make_async_copy(d1_hbm.at[dstsec.at[sl]], dbuf[p], gsem[p][1]).wait()

    def sissue(p):
        pltpu.async_copy(tbuf[p], acc.at[dstv[p]], ssem[p], add=True)

    def swait(p):
        pltpu.make_async_copy(tbuf[p], acc.at[dstv[p]], ssem[p]).wait()

    def compute(p):
        tb = tbuf[p]
        db = dbuf[p]

        @pl.loop(0, CHUNK)
        def _(b):
            al = tb[b, pl.ds(HF, 16)] + db[b, :]
            w16 = jnp.exp(_lrelu(al))
            tb[b, pl.ds(HF, 16)] = w16
            for hh in range(HEADS):
                tb[b, pl.ds(16 * hh, 16)] = tb[b, pl.ds(16 * hh, 16)] * w16[hh]

    for sec in range(NSEC):
        soff = base + sec * SECE
        pltpu.sync_copy(src_hbm.at[pl.ds(soff, SECE)], srcsec)
        pltpu.sync_copy(dst_hbm.at[pl.ds(soff, SECE)], dstsec)
        issue(0, 0)

        @pl.loop(0, SECC // 2)
        def _(k):
            l0 = 2 * k
            gwait(l0, 0)

            @pl.when(l0 >= 1)
            def _():
                swait(1)
            issue(l0 + 1, 1)
            compute(0)
            sissue(0)

            gwait(l0 + 1, 1)
            swait(0)

            @pl.when(l0 + 2 < SECC)
            def _():
                issue(l0 + 2, 0)
            compute(1)
            sissue(1)

        gwait(SECC - 1, 0)
        swait(1)
        compute(0)
        sissue(0)
        swait(0)

    plsc.subcore_barrier()
    pltpu.sync_copy(acc.at[pl.ds(row0, ZR)], p1_hbm.at[cid, pl.ds(row0, ZR)])

    @pl.when(sid == 0)
    def _():
        pltpu.sync_copy(acc.at[pl.ds(NS * ZR, ZREM)],
                        p1_hbm.at[cid, pl.ds(NS * ZR, ZREM)])


def _edge1(t1, d1, src, dst):
    mesh = plsc.VectorSubcoreMesh(core_axis_name="c", subcore_axis_name="s")
    return pl.kernel(
        _edge1_body,
        out_type=jax.ShapeDtypeStruct((NC, N, TW), jnp.float32),
        mesh=mesh,
        compiler_params=pltpu.CompilerParams(use_tc_tiling_on_sc=False,
                                             needs_layout_passes=False),
        scratch_types=(
            [pltpu.VMEM_SHARED((N, TW), jnp.float32)]
            + [pltpu.VMEM((SECE,), jnp.int32)] * 2
            + [pltpu.VMEM((CHUNK,), jnp.int32)] * 2
            + [pltpu.VMEM((CHUNK, TW), jnp.float32)] * 2
            + [pltpu.VMEM((CHUNK, 16), jnp.float32)] * 2
            + [pltpu.SemaphoreType.DMA] * 6
        ),
    )(t1, d1, src, dst)


def _stage_c_body(p10_ref, p11_ref, t1_ref, d1_ref, b1_ref, w2_ref,
                  as2_ref, ad2_ref, t2_ref):
    t1 = t1_ref[...]
    h1 = t1[:, :HF]
    asrc = t1[:, HF:HF + HEADS]
    adst = d1_ref[...][:, :HEADS]
    wself = jnp.exp(_lrelu(asrc + adst))
    wself_x = jnp.broadcast_to(wself[:, :, None], (ROW_BLK, HEADS, HID))
    wself_x = wself_x.reshape(ROW_BLK, HF)
    s = p10_ref[0][:, :HF] + p11_ref[0][:, :HF] + wself_x * h1
    den = (p10_ref[0][:, HF:HF + HEADS] + p11_ref[0][:, HF:HF + HEADS]
           + wself)
    den_x = jnp.broadcast_to(den[:, :, None], (ROW_BLK, HEADS, HID))
    den_x = den_x.reshape(ROW_BLK, HF)
    o1 = s / (den_x + 1e-16) + b1_ref[...][0]
    e1 = jnp.where(o1 > 0, o1, jnp.exp(o1) - 1.0)
    h2 = jnp.dot(e1, w2_ref[...], preferred_element_type=jnp.float32)
    a20 = as2_ref[0, 0]
    a21 = as2_ref[0, 1]
    b20 = ad2_ref[0, 0]
    b21 = ad2_ref[0, 1]
    asrc2 = h2[:, 0] * a20 + h2[:, 1] * a21
    adst2 = h2[:, 0] * b20 + h2[:, 1] * b21
    t2_ref[...] = jnp.stack([h2[:, 0], h2[:, 1], asrc2, adst2], axis=1)


def _stage_c(p1, t1, d1, b1, w2, as2, ad2):
    return pl.pallas_call(
        _stage_c_body,
        grid=(GRID,),
        in_specs=[
            pl.BlockSpec((1, ROW_BLK, TW), lambda i: (0, i, 0)),
            pl.BlockSpec((1, ROW_BLK, TW), lambda i: (1, i, 0)),
            pl.BlockSpec((ROW_BLK, TW), lambda i: (i, 0)),
            pl.BlockSpec((ROW_BLK, 16), lambda i: (i, 0)),
            pl.BlockSpec((1, HF), lambda i: (0, 0)),
            pl.BlockSpec((HF, OUT), lambda i: (0, 0)),
            pl.BlockSpec((1, OUT), lambda i: (0, 0)),
            pl.BlockSpec((1, OUT), lambda i: (0, 0)),
        ],
        out_specs=pl.BlockSpec((ROW_BLK, 4), lambda i: (i, 0)),
        out_shape=jax.ShapeDtypeStruct((N, 4), jnp.float32),
    )(p1, p1, t1, d1, b1, w2, as2, ad2)


def _edge2_body(t2_hbm, src_hbm, dst_hbm, p2_hbm,
                acc2, t2v, srcall, dstall):
    cid = lax.axis_index("c")
    sid = lax.axis_index("s")
    wid = sid * NC + cid
    base = wid * EPW

    @pl.loop(0, N * 4 // 16)
    def _(b):
        acc2[pl.ds(16 * b, 16)] = jnp.zeros((16,), jnp.float32)

    pltpu.sync_copy(t2_hbm, t2v)
    pltpu.sync_copy(src_hbm.at[pl.ds(base, EPW)], srcall)
    pltpu.sync_copy(dst_hbm.at[pl.ds(base, EPW)], dstall)

    c0 = jnp.full((16,), 0, jnp.int32)
    c1 = jnp.full((16,), 1, jnp.int32)
    c2 = jnp.full((16,), 2, jnp.int32)
    c3 = jnp.full((16,), 3, jnp.int32)

    @pl.loop(0, EPW // 16)
    def _(g):
        sl = pl.ds(g * 16, 16)
        s4 = srcall[sl] * 4
        d4 = dstall[sl] * 4
        sa = plsc.load_gather(t2v, [s4 + c2])
        da = plsc.load_gather(t2v, [d4 + c3])
        w = jnp.exp(_lrelu(sa + da))
        m0 = plsc.load_gather(t2v, [s4 + c0]) * w
        m1 = plsc.load_gather(t2v, [s4 + c1]) * w
        plsc.addupdate_scatter(acc2, [d4 + c0], m0)
        plsc.addupdate_scatter(acc2, [d4 + c1], m1)
        plsc.addupdate_scatter(acc2, [d4 + c2], w)

    pltpu.sync_copy(acc2, p2_hbm.at[wid])


def _edge2(t2flat, src, dst):
    mesh = plsc.VectorSubcoreMesh(core_axis_name="c", subcore_axis_name="s")
    return pl.kernel(
        _edge2_body,
        out_type=jax.ShapeDtypeStruct((NW, N * 4), jnp.float32),
        mesh=mesh,
        compiler_params=pltpu.CompilerParams(use_tc_tiling_on_sc=False,
                                             needs_layout_passes=False),
        scratch_types=[
            pltpu.VMEM((N * 4,), jnp.float32),
            pltpu.VMEM((N * 4,), jnp.float32),
            pltpu.VMEM((EPW,), jnp.int32),
            pltpu.VMEM((EPW,), jnp.int32),
        ],
    )(t2flat, src, dst)


def _reduce_body(p_ref, o_ref):
    o_ref[...] = jnp.sum(p_ref[...], axis=0)


def _reduce_partials(p2flat):
    return pl.pallas_call(
        _reduce_body,
        out_shape=jax.ShapeDtypeStruct((N * 4,), jnp.float32),
    )(p2flat)


def _stage_e_body(p2_ref, t2_ref, b2_ref, out_ref):
    t2 = t2_ref[...]
    p2 = p2_ref[...]
    wself = jnp.exp(_lrelu(t2[:, 2] + t2[:, 3]))
    s0 = p2[:, 0] + wself * t2[:, 0]
    s1 = p2[:, 1] + wself * t2[:, 1]
    den = p2[:, 2] + wself + 1e-16
    out_ref[...] = (jnp.stack([s0, s1], axis=1) / den[:, None]
                    + b2_ref[...][0])


def _stage_e(p2, t2, b2):
    return pl.pallas_call(
        _stage_e_body,
        grid=(GRID,),
        in_specs=[
            pl.BlockSpec((ROW_BLK, 4), lambda i: (i, 0)),
            pl.BlockSpec((ROW_BLK, 4), lambda i: (i, 0)),
            pl.BlockSpec((1, OUT), lambda i: (0, 0)),
        ],
        out_specs=pl.BlockSpec((ROW_BLK, OUT), lambda i: (i, 0)),
        out_shape=jax.ShapeDtypeStruct((N, OUT), jnp.float32),
    )(p2, t2, b2)


def kernel(x, edge_index, W1, att_src1, att_dst1, b1, W2, att_src2, att_dst2, b2):
    src = edge_index[0]
    dst = edge_index[1]
    eye = jnp.eye(HEADS, dtype=jnp.float32)
    a_s = (att_src1[:, :, None] * eye[:, None, :]).reshape(HF, HEADS)
    a_d = (att_dst1[:, :, None] * eye[:, None, :]).reshape(HF, HEADS)

    t1, d1 = _stage_a(x, W1, a_s, a_d)
    p1 = _edge1(t1, d1, src, dst)
    t2 = _stage_c(p1, t1, d1, b1.reshape(1, HF), W2, att_src2, att_dst2)
    p2 = _edge2(t2.reshape(N * 4), src, dst)
    p2s = _reduce_partials(p2)
    return _stage_e(p2s.reshape(N, 4), t2, b2.reshape(1, OUT))


# speedup vs baseline: 1.4316x; 1.0106x over previous
import jax
import jax.numpy as jnp
from jax import lax
from jax.experimental import pallas as pl
from jax.experimental.pallas import tpu as pltpu
from jax.experimental.pallas import tpu_sc as plsc

N = 10000
E = 320000
F_IN = 128
HID = 16
HEADS = 8
HF = HEADS * HID
TW = HF + 16
OUT = 2

NC = 2
NS = 16
NW = NC * NS
EPW = E // NW
CHUNK = 80
NSEC = 5
SECE = EPW // NSEC
SECC = SECE // CHUNK
ZR = 624
ZREM = N - NS * ZR

ROW_BLK = 1000
GRID = N // ROW_BLK


def _lrelu(x):
    return jnp.where(x >= 0, x, 0.2 * x)


def _stage_a_body(x_ref, w1_ref, as_ref, ad_ref, t1_ref, d1_ref):
    h = jnp.dot(x_ref[...], w1_ref[...], preferred_element_type=jnp.float32)
    z = jnp.zeros((ROW_BLK, 8), jnp.float32)
    t1_ref[...] = jnp.concatenate(
        [h, jnp.dot(h, as_ref[...], preferred_element_type=jnp.float32), z],
        axis=1)
    d1_ref[...] = jnp.concatenate(
        [jnp.dot(h, ad_ref[...], preferred_element_type=jnp.float32), z], axis=1)


def _stage_a(x, w1, a_s, a_d):
    return pl.pallas_call(
        _stage_a_body,
        grid=(GRID,),
        in_specs=[
            pl.BlockSpec((ROW_BLK, F_IN), lambda i: (i, 0)),
            pl.BlockSpec((F_IN, HF), lambda i: (0, 0)),
            pl.BlockSpec((HF, HEADS), lambda i: (0, 0)),
            pl.BlockSpec((HF, HEADS), lambda i: (0, 0)),
        ],
        out_specs=[
            pl.BlockSpec((ROW_BLK, TW), lambda i: (i, 0)),
            pl.BlockSpec((ROW_BLK, 16), lambda i: (i, 0)),
        ],
        out_shape=[
            jax.ShapeDtypeStruct((N, TW), jnp.float32),
            jax.ShapeDtypeStruct((N, 16), jnp.float32),
        ],
    )(x, w1, a_s, a_d)


def _edge1_body(t1_hbm, d1_hbm, src_hbm, dst_hbm, p1_hbm,
                acc, srcsec0, srcsec1, dstsec0, dstsec1,
                dstv0, dstv1, tbuf0, tbuf1,
                dbuf0, dbuf1,
                gt0, gd0, gt1, gd1, ss0, ss1, is0, is1, id0, id1):
    cid = lax.axis_index("c")
    sid = lax.axis_index("s")
    wid = sid * NC + cid
    row0 = sid * ZR
    dstv = (dstv0, dstv1)
    tbuf = (tbuf0, tbuf1)
    dbuf = (dbuf0, dbuf1)
    srcsecs = (srcsec0, srcsec1)
    dstsecs = (dstsec0, dstsec1)
    gsem = ((gt0, gd0), (gt1, gd1))
    ssem = (ss0, ss1)
    isem = ((is0, id0), (is1, id1))

    @pl.loop(0, CHUNK)
    def _(b):
        for j in range(TW // 16):
            tbuf0[b, pl.ds(16 * j, 16)] = jnp.zeros((16,), jnp.float32)

    for j in range(ZR // CHUNK):
        pltpu.sync_copy(tbuf0.at[pl.ds(0, CHUNK)],
                        acc.at[pl.ds(row0 + j * CHUNK, CHUNK)])
    ztail = ZR - (ZR // CHUNK) * CHUNK
    pltpu.sync_copy(tbuf0.at[pl.ds(0, ztail)],
                    acc.at[pl.ds(row0 + ZR - ztail, ztail)])

    @pl.when(sid == 0)
    def _():
        pltpu.sync_copy(tbuf0.at[pl.ds(0, ZREM)], acc.at[pl.ds(NS * ZR, ZREM)])
    plsc.subcore_barrier()

    base = wid * EPW

    def sec_issue(sec, q):
        soff = base + sec * SECE
        pltpu.async_copy(src_hbm.at[pl.ds(soff, SECE)], srcsecs[q], isem[q][0])
        pltpu.async_copy(dst_hbm.at[pl.ds(soff, SECE)], dstsecs[q], isem[q][1])

    def sec_wait(sec, q):
        soff = base + sec * SECE
        pltpu.make_async_copy(src_hbm.at[pl.ds(soff, SECE)], srcsecs[q],
                              isem[q][0]).wait()
        pltpu.make_async_copy(dst_hbm.at[pl.ds(soff, SECE)], dstsecs[q],
                              isem[q][1]).wait()

    def issue(l, p, srcsec, dstsec):
        sl = pl.ds(l * CHUNK, CHUNK)
        for i in range(CHUNK // 16):
            dstv[p][pl.ds(16 * i, 16)] = dstsec[pl.ds(l * CHUNK + 16 * i, 16)]
        pltpu.async_copy(t1_hbm.at[srcsec.at[sl]], tbuf[p], gsem[p][0])
        pltpu.async_copy(d1_hbm.at[dstsec.at[sl]], dbuf[p], gsem[p][1])

    def gwait(l, p, srcsec, dstsec):
        sl = pl.ds(l * CHUNK, CHUNK)
        pltpu.make_async_copy(t1_hbm.at[srcsec.at[sl]], tbuf[p], gsem[p][0]).wait()
        pltpu.make_async_copy(d1_hbm.at[dstsec.at[sl]], dbuf[p], gsem[p][1]).wait()

    def sissue(p):
        pltpu.async_copy(tbuf[p], acc.at[dstv[p]], ssem[p], add=True)

    def swait(p):
        pltpu.make_async_copy(tbuf[p], acc.at[dstv[p]], ssem[p]).wait()

    def compute(p):
        tb = tbuf[p]
        db = dbuf[p]

        @pl.loop(0, CHUNK)
        def _(b):
            al = tb[b, pl.ds(HF, 16)] + db[b, :]
            w16 = jnp.exp(_lrelu(al))
            tb[b, pl.ds(HF, 16)] = w16
            for hh in range(HEADS):
                wbh = w16[jnp.full((16,), hh, jnp.int32)]
                tb[b, pl.ds(16 * hh, 16)] = tb[b, pl.ds(16 * hh, 16)] * wbh

    sec_issue(0, 0)
    for sec in range(NSEC):
        q = sec % 2
        srcsec = srcsecs[q]
        dstsec = dstsecs[q]
        sec_wait(sec, q)
        if sec + 1 < NSEC:
            sec_issue(sec + 1, 1 - q)
        issue(0, 0, srcsec, dstsec)

        @pl.loop(0, SECC // 2)
        def _(k):
            l0 = 2 * k
            gwait(l0, 0, srcsec, dstsec)

            @pl.when(l0 >= 1)
            def _():
                swait(1)
            issue(l0 + 1, 1, srcsec, dstsec)
            compute(0)
            sissue(0)

            gwait(l0 + 1, 1, srcsec, dstsec)
            swait(0)

            @pl.when(l0 + 2 < SECC)
            def _():
                issue(l0 + 2, 0, srcsec, dstsec)
            compute(1)
            sissue(1)

        gwait(SECC - 1, 0, srcsec, dstsec)
        swait(1)
        compute(0)
        sissue(0)
        swait(0)

    plsc.subcore_barrier()
    pltpu.sync_copy(acc.at[pl.ds(row0, ZR)], p1_hbm.at[cid, pl.ds(row0, ZR)])

    @pl.when(sid == 0)
    def _():
        pltpu.sync_copy(acc.at[pl.ds(NS * ZR, ZREM)],
                        p1_hbm.at[cid, pl.ds(NS * ZR, ZREM)])


def _edge1(t1, d1, src, dst):
    mesh = plsc.VectorSubcoreMesh(core_axis_name="c", subcore_axis_name="s")
    return pl.kernel(
        _edge1_body,
        out_type=jax.ShapeDtypeStruct((NC, N, TW), jnp.float32),
        mesh=mesh,
        compiler_params=pltpu.CompilerParams(use_tc_tiling_on_sc=False,
                                             needs_layout_passes=False),
        scratch_types=(
            [pltpu.VMEM_SHARED((N, TW), jnp.float32)]
            + [pltpu.VMEM((SECE,), jnp.int32)] * 4
            + [pltpu.VMEM((CHUNK,), jnp.int32)] * 2
            + [pltpu.VMEM((CHUNK, TW), jnp.float32)] * 2
            + [pltpu.VMEM((CHUNK, 16), jnp.float32)] * 2
            + [pltpu.SemaphoreType.DMA] * 10
        ),
    )(t1, d1, src, dst)


def _stage_c_body(p10_ref, p11_ref, t1_ref, d1_ref, b1_ref, w2_ref,
                  as2_ref, ad2_ref, t2_ref):
    t1 = t1_ref[...]
    h1 = t1[:, :HF]
    asrc = t1[:, HF:HF + HEADS]
    adst = d1_ref[...][:, :HEADS]
    wself = jnp.exp(_lrelu(asrc + adst))
    wself_x = jnp.broadcast_to(wself[:, :, None], (ROW_BLK, HEADS, HID))
    wself_x = wself_x.reshape(ROW_BLK, HF)
    s = p10_ref[0][:, :HF] + p11_ref[0][:, :HF] + wself_x * h1
    den = (p10_ref[0][:, HF:HF + HEADS] + p11_ref[0][:, HF:HF + HEADS]
           + wself)
    den_x = jnp.broadcast_to(den[:, :, None], (ROW_BLK, HEADS, HID))
    den_x = den_x.reshape(ROW_BLK, HF)
    o1 = s / (den_x + 1e-16) + b1_ref[...][0]
    e1 = jnp.where(o1 > 0, o1, jnp.exp(o1) - 1.0)
    h2 = jnp.dot(e1, w2_ref[...], preferred_element_type=jnp.float32)
    a20 = as2_ref[0, 0]
    a21 = as2_ref[0, 1]
    b20 = ad2_ref[0, 0]
    b21 = ad2_ref[0, 1]
    asrc2 = h2[:, 0] * a20 + h2[:, 1] * a21
    adst2 = h2[:, 0] * b20 + h2[:, 1] * b21
    t2_ref[...] = jnp.stack([h2[:, 0], h2[:, 1], asrc2, adst2], axis=1)


def _stage_c(p1, t1, d1, b1, w2, as2, ad2):
    return pl.pallas_call(
        _stage_c_body,
        grid=(GRID,),
        in_specs=[
            pl.BlockSpec((1, ROW_BLK, TW), lambda i: (0, i, 0)),
            pl.BlockSpec((1, ROW_BLK, TW), lambda i: (1, i, 0)),
            pl.BlockSpec((ROW_BLK, TW), lambda i: (i, 0)),
            pl.BlockSpec((ROW_BLK, 16), lambda i: (i, 0)),
            pl.BlockSpec((1, HF), lambda i: (0, 0)),
            pl.BlockSpec((HF, OUT), lambda i: (0, 0)),
            pl.BlockSpec((1, OUT), lambda i: (0, 0)),
            pl.BlockSpec((1, OUT), lambda i: (0, 0)),
        ],
        out_specs=pl.BlockSpec((ROW_BLK, 4), lambda i: (i, 0)),
        out_shape=jax.ShapeDtypeStruct((N, 4), jnp.float32),
    )(p1, p1, t1, d1, b1, w2, as2, ad2)


def _edge2_body(t2_hbm, src_hbm, dst_hbm, p2_hbm,
                acc2, t2v, srcall, dstall):
    cid = lax.axis_index("c")
    sid = lax.axis_index("s")
    wid = sid * NC + cid
    base = wid * EPW

    @pl.loop(0, N * 4 // 16)
    def _(b):
        acc2[pl.ds(16 * b, 16)] = jnp.zeros((16,), jnp.float32)

    pltpu.sync_copy(t2_hbm, t2v)
    pltpu.sync_copy(src_hbm.at[pl.ds(base, EPW)], srcall)
    pltpu.sync_copy(dst_hbm.at[pl.ds(base, EPW)], dstall)

    c0 = jnp.full((16,), 0, jnp.int32)
    c1 = jnp.full((16,), 1, jnp.int32)
    c2 = jnp.full((16,), 2, jnp.int32)
    c3 = jnp.full((16,), 3, jnp.int32)

    @pl.loop(0, EPW // 16)
    def _(g):
        sl = pl.ds(g * 16, 16)
        s4 = srcall[sl] * 4
        d4 = dstall[sl] * 4
        sa = plsc.load_gather(t2v, [s4 + c2])
        da = plsc.load_gather(t2v, [d4 + c3])
        w = jnp.exp(_lrelu(sa + da))
        m0 = plsc.load_gather(t2v, [s4 + c0]) * w
        m1 = plsc.load_gather(t2v, [s4 + c1]) * w
        plsc.addupdate_scatter(acc2, [d4 + c0], m0)
        plsc.addupdate_scatter(acc2, [d4 + c1], m1)
        plsc.addupdate_scatter(acc2, [d4 + c2], w)

    pltpu.sync_copy(acc2, p2_hbm.at[wid])


def _edge2(t2flat, src, dst):
    mesh = plsc.VectorSubcoreMesh(core_axis_name="c", subcore_axis_name="s")
    return pl.kernel(
        _edge2_body,
        out_type=jax.ShapeDtypeStruct((NW, N * 4), jnp.float32),
        mesh=mesh,
        compiler_params=pltpu.CompilerParams(use_tc_tiling_on_sc=False,
                                             needs_layout_passes=False),
        scratch_types=[
            pltpu.VMEM((N * 4,), jnp.float32),
            pltpu.VMEM((N * 4,), jnp.float32),
            pltpu.VMEM((EPW,), jnp.int32),
            pltpu.VMEM((EPW,), jnp.int32),
        ],
    )(t2flat, src, dst)


def _reduce_body(p_ref, o_ref):
    o_ref[...] = jnp.sum(p_ref[...], axis=0)


def _reduce_partials(p2flat):
    return pl.pallas_call(
        _reduce_body,
        out_shape=jax.ShapeDtypeStruct((N * 4,), jnp.float32),
    )(p2flat)


def _stage_e_body(p2_ref, t2_ref, b2_ref, out_ref):
    t2 = t2_ref[...]
    p2 = p2_ref[...]
    wself = jnp.exp(_lrelu(t2[:, 2] + t2[:, 3]))
    s0 = p2[:, 0] + wself * t2[:, 0]
    s1 = p2[:, 1] + wself * t2[:, 1]
    den = p2[:, 2] + wself + 1e-16
    out_ref[...] = (jnp.stack([s0, s1], axis=1) / den[:, None]
                    + b2_ref[...][0])


def _stage_e(p2, t2, b2):
    return pl.pallas_call(
        _stage_e_body,
        grid=(GRID,),
        in_specs=[
            pl.BlockSpec((ROW_BLK, 4), lambda i: (i, 0)),
            pl.BlockSpec((ROW_BLK, 4), lambda i: (i, 0)),
            pl.BlockSpec((1, OUT), lambda i: (0, 0)),
        ],
        out_specs=pl.BlockSpec((ROW_BLK, OUT), lambda i: (i, 0)),
        out_shape=jax.ShapeDtypeStruct((N, OUT), jnp.float32),
    )(p2, t2, b2)


def kernel(x, edge_index, W1, att_src1, att_dst1, b1, W2, att_src2, att_dst2, b2):
    src = edge_index[0]
    dst = edge_index[1]
    eye = jnp.eye(HEADS, dtype=jnp.float32)
    a_s = (att_src1[:, :, None] * eye[:, None, :]).reshape(HF, HEADS)
    a_d = (att_dst1[:, :, None] * eye[:, None, :]).reshape(HF, HEADS)

    t1, d1 = _stage_a(x, W1, a_s, a_d)
    p1 = _edge1(t1, d1, src, dst)
    t2 = _stage_c(p1, t1, d1, b1.reshape(1, HF), W2, att_src2, att_dst2)
    p2 = _edge2(t2.reshape(N * 4), src, dst)
    p2s = _reduce_partials(p2)
    return _stage_e(p2s.reshape(N, 4), t2, b2.reshape(1, OUT))


# speedup vs baseline: 1.4327x; 1.0007x over previous
import jax
import jax.numpy as jnp
from jax import lax
from jax.experimental import pallas as pl
from jax.experimental.pallas import tpu as pltpu
from jax.experimental.pallas import tpu_sc as plsc

N = 10000
E = 320000
F_IN = 128
HID = 16
HEADS = 8
HF = HEADS * HID
TW = HF + 16
OUT = 2

NC = 2
NS = 16
NW = NC * NS
EPW = E // NW
CHUNK = 80
NSEC = 5
SECE = EPW // NSEC
SECC = SECE // CHUNK
ZR = 624
ZREM = N - NS * ZR

ROW_BLK = 1000
GRID = N // ROW_BLK


def _lrelu(x):
    return jnp.where(x >= 0, x, 0.2 * x)


def _stage_a_body(x_ref, w1_ref, as_ref, ad_ref, t1_ref, d1_ref):
    h = jnp.dot(x_ref[...], w1_ref[...], preferred_element_type=jnp.float32)
    z = jnp.zeros((ROW_BLK, 8), jnp.float32)
    t1_ref[...] = jnp.concatenate(
        [h, jnp.dot(h, as_ref[...], preferred_element_type=jnp.float32), z],
        axis=1)
    d1_ref[...] = jnp.concatenate(
        [jnp.dot(h, ad_ref[...], preferred_element_type=jnp.float32), z], axis=1)


def _stage_a(x, w1, a_s, a_d):
    return pl.pallas_call(
        _stage_a_body,
        grid=(GRID,),
        in_specs=[
            pl.BlockSpec((ROW_BLK, F_IN), lambda i: (i, 0)),
            pl.BlockSpec((F_IN, HF), lambda i: (0, 0)),
            pl.BlockSpec((HF, HEADS), lambda i: (0, 0)),
            pl.BlockSpec((HF, HEADS), lambda i: (0, 0)),
        ],
        out_specs=[
            pl.BlockSpec((ROW_BLK, TW), lambda i: (i, 0)),
            pl.BlockSpec((ROW_BLK, 16), lambda i: (i, 0)),
        ],
        out_shape=[
            jax.ShapeDtypeStruct((N, TW), jnp.float32),
            jax.ShapeDtypeStruct((N, 16), jnp.float32),
        ],
    )(x, w1, a_s, a_d)


def _edge1_body(t1_hbm, d1_hbm, src_hbm, dst_hbm, p1_hbm,
                acc, srcsec0, srcsec1, dstsec0, dstsec1,
                dstv0, dstv1, tbuf0, tbuf1,
                dbuf0, dbuf1,
                gt0, gd0, gt1, gd1, ss0, ss1, is0, is1, id0, id1):
    cid = lax.axis_index("c")
    sid = lax.axis_index("s")
    wid = sid * NC + cid
    row0 = sid * ZR
    dstv = (dstv0, dstv1)
    tbuf = (tbuf0, tbuf1)
    dbuf = (dbuf0, dbuf1)
    srcsecs = (srcsec0, srcsec1)
    dstsecs = (dstsec0, dstsec1)
    gsem = ((gt0, gd0), (gt1, gd1))
    ssem = (ss0, ss1)
    isem = ((is0, id0), (is1, id1))

    @pl.loop(0, CHUNK)
    def _(b):
        for j in range(TW // 16):
            tbuf0[b, pl.ds(16 * j, 16)] = jnp.zeros((16,), jnp.float32)

    for j in range(ZR // CHUNK):
        pltpu.sync_copy(tbuf0.at[pl.ds(0, CHUNK)],
                        acc.at[pl.ds(row0 + j * CHUNK, CHUNK)])
    ztail = ZR - (ZR // CHUNK) * CHUNK
    pltpu.sync_copy(tbuf0.at[pl.ds(0, ztail)],
                    acc.at[pl.ds(row0 + ZR - ztail, ztail)])

    @pl.when(sid == 0)
    def _():
        pltpu.sync_copy(tbuf0.at[pl.ds(0, ZREM)], acc.at[pl.ds(NS * ZR, ZREM)])
    plsc.subcore_barrier()

    base = wid * EPW

    def sec_issue(sec, q):
        soff = base + sec * SECE
        pltpu.async_copy(src_hbm.at[pl.ds(soff, SECE)], srcsecs[q], isem[q][0])
        pltpu.async_copy(dst_hbm.at[pl.ds(soff, SECE)], dstsecs[q], isem[q][1])

    def sec_wait(sec, q):
        soff = base + sec * SECE
        pltpu.make_async_copy(src_hbm.at[pl.ds(soff, SECE)], srcsecs[q],
                              isem[q][0]).wait()
        pltpu.make_async_copy(dst_hbm.at[pl.ds(soff, SECE)], dstsecs[q],
                              isem[q][1]).wait()

    def issue(l, p, srcsec, dstsec):
        sl = pl.ds(l * CHUNK, CHUNK)
        for i in range(CHUNK // 16):
            dstv[p][pl.ds(16 * i, 16)] = dstsec[pl.ds(l * CHUNK + 16 * i, 16)]
        pltpu.async_copy(t1_hbm.at[srcsec.at[sl]], tbuf[p], gsem[p][0])
        pltpu.async_copy(d1_hbm.at[dstsec.at[sl]], dbuf[p], gsem[p][1])

    def gwait(l, p, srcsec, dstsec):
        sl = pl.ds(l * CHUNK, CHUNK)
        pltpu.make_async_copy(t1_hbm.at[srcsec.at[sl]], tbuf[p], gsem[p][0]).wait()
        pltpu.make_async_copy(d1_hbm.at[dstsec.at[sl]], dbuf[p], gsem[p][1]).wait()

    def sissue(p):
        pltpu.async_copy(tbuf[p], acc.at[dstv[p]], ssem[p], add=True)

    def swait(p):
        pltpu.make_async_copy(tbuf[p], acc.at[dstv[p]], ssem[p]).wait()

    def compute(p):
        tb = tbuf[p]
        db = dbuf[p]

        @pl.loop(0, CHUNK)
        def _(b):
            al = tb[b, pl.ds(HF, 16)] + db[b, :]
            w16 = jnp.exp(_lrelu(al))
            tb[b, pl.ds(HF, 16)] = w16
            for hh in range(HEADS):
                tb[b, pl.ds(16 * hh, 16)] = tb[b, pl.ds(16 * hh, 16)] * w16[hh]

    sec_issue(0, 0)
    for sec in range(NSEC):
        q = sec % 2
        srcsec = srcsecs[q]
        dstsec = dstsecs[q]
        sec_wait(sec, q)
        if sec + 1 < NSEC:
            sec_issue(sec + 1, 1 - q)
        issue(0, 0, srcsec, dstsec)

        @pl.loop(0, SECC // 2)
        def _(k):
            l0 = 2 * k
            gwait(l0, 0, srcsec, dstsec)

            @pl.when(l0 >= 1)
            def _():
                swait(1)
            issue(l0 + 1, 1, srcsec, dstsec)
            compute(0)
            sissue(0)

            gwait(l0 + 1, 1, srcsec, dstsec)
            swait(0)

            @pl.when(l0 + 2 < SECC)
            def _():
                issue(l0 + 2, 0, srcsec, dstsec)
            compute(1)
            sissue(1)

        gwait(SECC - 1, 0, srcsec, dstsec)
        swait(1)
        compute(0)
        sissue(0)
        swait(0)

    plsc.subcore_barrier()
    pltpu.sync_copy(acc.at[pl.ds(row0, ZR)], p1_hbm.at[cid, pl.ds(row0, ZR)])

    @pl.when(sid == 0)
    def _():
        pltpu.sync_copy(acc.at[pl.ds(NS * ZR, ZREM)],
                        p1_hbm.at[cid, pl.ds(NS * ZR, ZREM)])


def _edge1(t1, d1, src, dst):
    mesh = plsc.VectorSubcoreMesh(core_axis_name="c", subcore_axis_name="s")
    return pl.kernel(
        _edge1_body,
        out_type=jax.ShapeDtypeStruct((NC, N, TW), jnp.float32),
        mesh=mesh,
        compiler_params=pltpu.CompilerParams(use_tc_tiling_on_sc=False,
                                             needs_layout_passes=False),
        scratch_types=(
            [pltpu.VMEM_SHARED((N, TW), jnp.float32)]
            + [pltpu.VMEM((SECE,), jnp.int32)] * 4
            + [pltpu.VMEM((CHUNK,), jnp.int32)] * 2
            + [pltpu.VMEM((CHUNK, TW), jnp.float32)] * 2
            + [pltpu.VMEM((CHUNK, 16), jnp.float32)] * 2
            + [pltpu.SemaphoreType.DMA] * 10
        ),
    )(t1, d1, src, dst)


def _stage_c_body(p10_ref, p11_ref, t1_ref, d1_ref, b1_ref, w2_ref,
                  as2_ref, ad2_ref, t2_ref):
    t1 = t1_ref[...]
    h1 = t1[:, :HF]
    asrc = t1[:, HF:HF + HEADS]
    adst = d1_ref[...][:, :HEADS]
    wself = jnp.exp(_lrelu(asrc + adst))
    wself_x = jnp.broadcast_to(wself[:, :, None], (ROW_BLK, HEADS, HID))
    wself_x = wself_x.reshape(ROW_BLK, HF)
    s = p10_ref[0][:, :HF] + p11_ref[0][:, :HF] + wself_x * h1
    den = (p10_ref[0][:, HF:HF + HEADS] + p11_ref[0][:, HF:HF + HEADS]
           + wself)
    den_x = jnp.broadcast_to(den[:, :, None], (ROW_BLK, HEADS, HID))
    den_x = den_x.reshape(ROW_BLK, HF)
    o1 = s / (den_x + 1e-16) + b1_ref[...][0]
    e1 = jnp.where(o1 > 0, o1, jnp.exp(o1) - 1.0)
    h2 = jnp.dot(e1, w2_ref[...], preferred_element_type=jnp.float32)
    a20 = as2_ref[0, 0]
    a21 = as2_ref[0, 1]
    b20 = ad2_ref[0, 0]
    b21 = ad2_ref[0, 1]
    asrc2 = h2[:, 0] * a20 + h2[:, 1] * a21
    adst2 = h2[:, 0] * b20 + h2[:, 1] * b21
    t2_ref[...] = jnp.stack([h2[:, 0], h2[:, 1], asrc2, adst2], axis=1)


def _stage_c(p1, t1, d1, b1, w2, as2, ad2):
    return pl.pallas_call(
        _stage_c_body,
        grid=(GRID,),
        in_specs=[
            pl.BlockSpec((1, ROW_BLK, TW), lambda i: (0, i, 0)),
            pl.BlockSpec((1, ROW_BLK, TW), lambda i: (1, i, 0)),
            pl.BlockSpec((ROW_BLK, TW), lambda i: (i, 0)),
            pl.BlockSpec((ROW_BLK, 16), lambda i: (i, 0)),
            pl.BlockSpec((1, HF), lambda i: (0, 0)),
            pl.BlockSpec((HF, OUT), lambda i: (0, 0)),
            pl.BlockSpec((1, OUT), lambda i: (0, 0)),
            pl.BlockSpec((1, OUT), lambda i: (0, 0)),
        ],
        out_specs=pl.BlockSpec((ROW_BLK, 4), lambda i: (i, 0)),
        out_shape=jax.ShapeDtypeStruct((N, 4), jnp.float32),
    )(p1, p1, t1, d1, b1, w2, as2, ad2)


def _edge2_body(t2_hbm, src_hbm, dst_hbm, p2_hbm,
                acc2, t2v, srcall, dstall):
    cid = lax.axis_index("c")
    sid = lax.axis_index("s")
    wid = sid * NC + cid
    base = wid * EPW

    @pl.loop(0, N * 4 // 16)
    def _(b):
        acc2[pl.ds(16 * b, 16)] = jnp.zeros((16,), jnp.float32)

    pltpu.sync_copy(t2_hbm, t2v)
    pltpu.sync_copy(src_hbm.at[pl.ds(base, EPW)], srcall)
    pltpu.sync_copy(dst_hbm.at[pl.ds(base, EPW)], dstall)

    c0 = jnp.full((16,), 0, jnp.int32)
    c1 = jnp.full((16,), 1, jnp.int32)
    c2 = jnp.full((16,), 2, jnp.int32)
    c3 = jnp.full((16,), 3, jnp.int32)

    @pl.loop(0, EPW // 16)
    def _(g):
        sl = pl.ds(g * 16, 16)
        s4 = srcall[sl] * 4
        d4 = dstall[sl] * 4
        sa = plsc.load_gather(t2v, [s4 + c2])
        da = plsc.load_gather(t2v, [d4 + c3])
        w = jnp.exp(_lrelu(sa + da))
        m0 = plsc.load_gather(t2v, [s4 + c0]) * w
        m1 = plsc.load_gather(t2v, [s4 + c1]) * w
        plsc.addupdate_scatter(acc2, [d4 + c0], m0)
        plsc.addupdate_scatter(acc2, [d4 + c1], m1)
        plsc.addupdate_scatter(acc2, [d4 + c2], w)

    pltpu.sync_copy(acc2, p2_hbm.at[wid])


def _edge2(t2flat, src, dst):
    mesh = plsc.VectorSubcoreMesh(core_axis_name="c", subcore_axis_name="s")
    return pl.kernel(
        _edge2_body,
        out_type=jax.ShapeDtypeStruct((NW, N * 4), jnp.float32),
        mesh=mesh,
        compiler_params=pltpu.CompilerParams(use_tc_tiling_on_sc=False,
                                             needs_layout_passes=False),
        scratch_types=[
            pltpu.VMEM((N * 4,), jnp.float32),
            pltpu.VMEM((N * 4,), jnp.float32),
            pltpu.VMEM((EPW,), jnp.int32),
            pltpu.VMEM((EPW,), jnp.int32),
        ],
    )(t2flat, src, dst)


def _reduce_body(p_ref, o_ref):
    o_ref[...] = jnp.sum(p_ref[...], axis=0)


def _reduce_partials(p2flat):
    return pl.pallas_call(
        _reduce_body,
        out_shape=jax.ShapeDtypeStruct((N * 4,), jnp.float32),
    )(p2flat)


def _stage_e_body(p2_ref, t2_ref, b2_ref, out_ref):
    t2 = t2_ref[...]
    p2 = p2_ref[...]
    wself = jnp.exp(_lrelu(t2[:, 2] + t2[:, 3]))
    s0 = p2[:, 0] + wself * t2[:, 0]
    s1 = p2[:, 1] + wself * t2[:, 1]
    den = p2[:, 2] + wself + 1e-16
    out_ref[...] = (jnp.stack([s0, s1], axis=1) / den[:, None]
                    + b2_ref[...][0])


def _stage_e(p2, t2, b2):
    return pl.pallas_call(
        _stage_e_body,
        grid=(GRID,),
        in_specs=[
            pl.BlockSpec((ROW_BLK, 4), lambda i: (i, 0)),
            pl.BlockSpec((ROW_BLK, 4), lambda i: (i, 0)),
            pl.BlockSpec((1, OUT), lambda i: (0, 0)),
        ],
        out_specs=pl.BlockSpec((ROW_BLK, OUT), lambda i: (i, 0)),
        out_shape=jax.ShapeDtypeStruct((N, OUT), jnp.float32),
    )(p2, t2, b2)


def kernel(x, edge_index, W1, att_src1, att_dst1, b1, W2, att_src2, att_dst2, b2):
    src = edge_index[0]
    dst = edge_index[1]
    eye = jnp.eye(HEADS, dtype=jnp.float32)
    a_s = (att_src1[:, :, None] * eye[:, None, :]).reshape(HF, HEADS)
    a_d = (att_dst1[:, :, None] * eye[:, None, :]).reshape(HF, HEADS)

    t1, d1 = _stage_a(x, W1, a_s, a_d)
    p1 = _edge1(t1, d1, src, dst)
    t2 = _stage_c(p1, t1, d1, b1.reshape(1, HF), W2, att_src2, att_dst2)
    p2 = _edge2(t2.reshape(N * 4), src, dst)
    p2s = _reduce_partials(p2)
    return _stage_e(p2s.reshape(N, 4), t2, b2.reshape(1, OUT))


# speedup vs baseline: 1.4426x; 1.0069x over previous
import jax
import jax.numpy as jnp
from jax import lax
from jax.experimental import pallas as pl
from jax.experimental.pallas import tpu as pltpu
from jax.experimental.pallas import tpu_sc as plsc

N = 10000
E = 320000
F_IN = 128
HID = 16
HEADS = 8
HF = HEADS * HID
TW = HF + 16
OUT = 2

NC = 2
NS = 16
NW = NC * NS
EPW = E // NW
CHUNK = 80
NSEC = 5
SECE = EPW // NSEC
SECC = SECE // CHUNK
ZR = 624
ZREM = N - NS * ZR

ROW_BLK = 1000
GRID = N // ROW_BLK


def _lrelu(x):
    return jnp.where(x >= 0, x, 0.2 * x)


def _stage_a_body(x_ref, w1_ref, as_ref, ad_ref, t1_ref, d1_ref):
    h = jnp.dot(x_ref[...], w1_ref[...], preferred_element_type=jnp.float32)
    z = jnp.zeros((ROW_BLK, 8), jnp.float32)
    t1_ref[...] = jnp.concatenate(
        [h, jnp.dot(h, as_ref[...], preferred_element_type=jnp.float32), z],
        axis=1)
    d1_ref[...] = jnp.concatenate(
        [jnp.dot(h, ad_ref[...], preferred_element_type=jnp.float32), z], axis=1)


def _stage_a(x, w1, a_s, a_d):
    return pl.pallas_call(
        _stage_a_body,
        grid=(GRID,),
        in_specs=[
            pl.BlockSpec((ROW_BLK, F_IN), lambda i: (i, 0)),
            pl.BlockSpec((F_IN, HF), lambda i: (0, 0)),
            pl.BlockSpec((HF, HEADS), lambda i: (0, 0)),
            pl.BlockSpec((HF, HEADS), lambda i: (0, 0)),
        ],
        out_specs=[
            pl.BlockSpec((ROW_BLK, TW), lambda i: (i, 0)),
            pl.BlockSpec((ROW_BLK, 16), lambda i: (i, 0)),
        ],
        out_shape=[
            jax.ShapeDtypeStruct((N, TW), jnp.float32),
            jax.ShapeDtypeStruct((N, 16), jnp.float32),
        ],
    )(x, w1, a_s, a_d)


def _edge1_body(t1_hbm, d1_hbm, src_hbm, dst_hbm, p1_hbm,
                acc, srcsec0, srcsec1, dstsec0, dstsec1,
                dstv0, dstv1, tbuf0, tbuf1,
                dbuf0, dbuf1,
                gt0, gd0, gt1, gd1, ss0, ss1, is0, is1, id0, id1):
    cid = lax.axis_index("c")
    sid = lax.axis_index("s")
    wid = sid * NC + cid
    row0 = sid * ZR
    dstv = (dstv0, dstv1)
    tbuf = (tbuf0, tbuf1)
    dbuf = (dbuf0, dbuf1)
    srcsecs = (srcsec0, srcsec1)
    dstsecs = (dstsec0, dstsec1)
    gsem = ((gt0, gd0), (gt1, gd1))
    ssem = (ss0, ss1)
    isem = ((is0, id0), (is1, id1))

    @pl.loop(0, CHUNK)
    def _(b):
        for j in range(TW // 16):
            tbuf0[b, pl.ds(16 * j, 16)] = jnp.zeros((16,), jnp.float32)

    for j in range(ZR // CHUNK):
        pltpu.sync_copy(tbuf0.at[pl.ds(0, CHUNK)],
                        acc.at[pl.ds(row0 + j * CHUNK, CHUNK)])
    ztail = ZR - (ZR // CHUNK) * CHUNK
    pltpu.sync_copy(tbuf0.at[pl.ds(0, ztail)],
                    acc.at[pl.ds(row0 + ZR - ztail, ztail)])

    @pl.when(sid == 0)
    def _():
        pltpu.sync_copy(tbuf0.at[pl.ds(0, ZREM)], acc.at[pl.ds(NS * ZR, ZREM)])
    plsc.subcore_barrier()

    base = wid * EPW

    def sec_issue(sec, q):
        soff = base + sec * SECE
        pltpu.async_copy(src_hbm.at[pl.ds(soff, SECE)], srcsecs[q], isem[q][0])
        pltpu.async_copy(dst_hbm.at[pl.ds(soff, SECE)], dstsecs[q], isem[q][1])

    def sec_wait(sec, q):
        soff = base + sec * SECE
        pltpu.make_async_copy(src_hbm.at[pl.ds(soff, SECE)], srcsecs[q],
                              isem[q][0]).wait()
        pltpu.make_async_copy(dst_hbm.at[pl.ds(soff, SECE)], dstsecs[q],
                              isem[q][1]).wait()

    def issue(l, p, srcsec, dstsec):
        sl = pl.ds(l * CHUNK, CHUNK)
        for i in range(CHUNK // 16):
            dstv[p][pl.ds(16 * i, 16)] = dstsec[pl.ds(l * CHUNK + 16 * i, 16)]
        pltpu.async_copy(t1_hbm.at[srcsec.at[sl]], tbuf[p], gsem[p][0])
        pltpu.async_copy(d1_hbm.at[dstsec.at[sl]], dbuf[p], gsem[p][1])

    def gwait(l, p, srcsec, dstsec):
        sl = pl.ds(l * CHUNK, CHUNK)
        pltpu.make_async_copy(t1_hbm.at[srcsec.at[sl]], tbuf[p], gsem[p][0]).wait()
        pltpu.make_async_copy(d1_hbm.at[dstsec.at[sl]], dbuf[p], gsem[p][1]).wait()

    def sissue(p):
        pltpu.async_copy(tbuf[p], acc.at[dstv[p]], ssem[p], add=True)

    def swait(p):
        pltpu.make_async_copy(tbuf[p], acc.at[dstv[p]], ssem[p]).wait()

    def compute(p):
        tb = tbuf[p]
        db = dbuf[p]

        @pl.loop(0, CHUNK)
        def _(b):
            al = tb[b, pl.ds(HF, 16)] + db[b, :]
            w16 = jnp.exp(_lrelu(al))
            tb[b, pl.ds(HF, 16)] = w16
            for hh in range(HEADS):
                tb[b, pl.ds(16 * hh, 16)] = tb[b, pl.ds(16 * hh, 16)] * w16[hh]

    sec_issue(0, 0)
    for sec in range(NSEC):
        q = sec % 2
        srcsec = srcsecs[q]
        dstsec = dstsecs[q]
        sec_wait(sec, q)
        if sec + 1 < NSEC:
            sec_issue(sec + 1, 1 - q)
        issue(0, 0, srcsec, dstsec)

        @pl.loop(0, SECC // 2)
        def _(k):
            l0 = 2 * k
            gwait(l0, 0, srcsec, dstsec)

            @pl.when(l0 >= 1)
            def _():
                swait(1)
            issue(l0 + 1, 1, srcsec, dstsec)
            compute(0)
            sissue(0)

            gwait(l0 + 1, 1, srcsec, dstsec)
            swait(0)

            @pl.when(l0 + 2 < SECC)
            def _():
                issue(l0 + 2, 0, srcsec, dstsec)
            compute(1)
            sissue(1)

        gwait(SECC - 1, 0, srcsec, dstsec)
        swait(1)
        compute(0)
        sissue(0)
        swait(0)

    plsc.subcore_barrier()
    pltpu.sync_copy(acc.at[pl.ds(row0, ZR)], p1_hbm.at[cid, pl.ds(row0, ZR)])

    @pl.when(sid == 0)
    def _():
        pltpu.sync_copy(acc.at[pl.ds(NS * ZR, ZREM)],
                        p1_hbm.at[cid, pl.ds(NS * ZR, ZREM)])


def _edge1(t1, d1, src, dst):
    mesh = plsc.VectorSubcoreMesh(core_axis_name="c", subcore_axis_name="s")
    return pl.kernel(
        _edge1_body,
        out_type=jax.ShapeDtypeStruct((NC, N, TW), jnp.float32),
        mesh=mesh,
        compiler_params=pltpu.CompilerParams(use_tc_tiling_on_sc=False,
                                             needs_layout_passes=False),
        scratch_types=(
            [pltpu.VMEM_SHARED((N, TW), jnp.float32)]
            + [pltpu.VMEM((SECE,), jnp.int32)] * 4
            + [pltpu.VMEM((CHUNK,), jnp.int32)] * 2
            + [pltpu.VMEM((CHUNK, TW), jnp.float32)] * 2
            + [pltpu.VMEM((CHUNK, 16), jnp.float32)] * 2
            + [pltpu.SemaphoreType.DMA] * 10
        ),
    )(t1, d1, src, dst)


def _stage_c_body(p10_ref, p11_ref, t1_ref, d1_ref, b1_ref, w2_ref,
                  as2_ref, ad2_ref, t2_ref):
    t1 = t1_ref[...]
    h1 = t1[:, :HF]
    asrc = t1[:, HF:HF + HEADS]
    adst = d1_ref[...][:, :HEADS]
    wself = jnp.exp(_lrelu(asrc + adst))
    wself_x = jnp.broadcast_to(wself[:, :, None], (ROW_BLK, HEADS, HID))
    wself_x = wself_x.reshape(ROW_BLK, HF)
    s = p10_ref[0][:, :HF] + p11_ref[0][:, :HF] + wself_x * h1
    den = (p10_ref[0][:, HF:HF + HEADS] + p11_ref[0][:, HF:HF + HEADS]
           + wself)
    den_x = jnp.broadcast_to(den[:, :, None], (ROW_BLK, HEADS, HID))
    den_x = den_x.reshape(ROW_BLK, HF)
    o1 = s / (den_x + 1e-16) + b1_ref[...][0]
    e1 = jnp.where(o1 > 0, o1, jnp.exp(o1) - 1.0)
    h2 = jnp.dot(e1, w2_ref[...], preferred_element_type=jnp.float32)
    a20 = as2_ref[0, 0]
    a21 = as2_ref[0, 1]
    b20 = ad2_ref[0, 0]
    b21 = ad2_ref[0, 1]
    asrc2 = h2[:, 0] * a20 + h2[:, 1] * a21
    adst2 = h2[:, 0] * b20 + h2[:, 1] * b21
    t2_ref[...] = jnp.stack([h2[:, 0], h2[:, 1], asrc2, adst2], axis=1)


def _stage_c(p1, t1, d1, b1, w2, as2, ad2):
    return pl.pallas_call(
        _stage_c_body,
        grid=(GRID,),
        in_specs=[
            pl.BlockSpec((1, ROW_BLK, TW), lambda i: (0, i, 0)),
            pl.BlockSpec((1, ROW_BLK, TW), lambda i: (1, i, 0)),
            pl.BlockSpec((ROW_BLK, TW), lambda i: (i, 0)),
            pl.BlockSpec((ROW_BLK, 16), lambda i: (i, 0)),
            pl.BlockSpec((1, HF), lambda i: (0, 0)),
            pl.BlockSpec((HF, OUT), lambda i: (0, 0)),
            pl.BlockSpec((1, OUT), lambda i: (0, 0)),
            pl.BlockSpec((1, OUT), lambda i: (0, 0)),
        ],
        out_specs=pl.BlockSpec((ROW_BLK, 4), lambda i: (i, 0)),
        out_shape=jax.ShapeDtypeStruct((N, 4), jnp.float32),
    )(p1, p1, t1, d1, b1, w2, as2, ad2)


AR = 2560
ART = AR // NS
NTR = AR // 128


def _edge2_body(t2_hbm, src_hbm, dst_hbm, p2_hbm,
                shacc, acc2, t2v, srcall, dstall, ibuf, rsem):
    cid = lax.axis_index("c")
    sid = lax.axis_index("s")
    wid = sid * NC + cid
    base = wid * EPW
    lane = jnp.arange(16, dtype=jnp.int32)

    @pl.loop(0, AR)
    def _(b):
        acc2[b, :] = jnp.zeros((16,), jnp.float32)

    @pl.loop(0, NTR)
    def _(l):
        for i in range(8):
            ibuf[l, pl.ds(16 * i, 16)] = 128 * l + 16 * i + lane

    pltpu.sync_copy(acc2.at[pl.ds(0, ART)], shacc.at[pl.ds(sid * ART, ART)])

    pltpu.sync_copy(t2_hbm, t2v)
    pltpu.sync_copy(src_hbm.at[pl.ds(base, EPW)], srcall)
    pltpu.sync_copy(dst_hbm.at[pl.ds(base, EPW)], dstall)
    plsc.subcore_barrier()

    c0 = jnp.full((16,), 0, jnp.int32)
    c1 = jnp.full((16,), 1, jnp.int32)
    c2 = jnp.full((16,), 2, jnp.int32)
    c3 = jnp.full((16,), 3, jnp.int32)

    @pl.loop(0, EPW // 16)
    def _(g):
        sl = pl.ds(g * 16, 16)
        s4 = srcall[sl] * 4
        d16 = dstall[sl]
        dq = d16 >> 2
        dr = (d16 & 3) * 4
        sa = plsc.load_gather(t2v, [s4 + c2])
        da = plsc.load_gather(t2v, [(d16 * 4) + c3])
        w = jnp.exp(_lrelu(sa + da))
        m0 = plsc.load_gather(t2v, [s4 + c0]) * w
        m1 = plsc.load_gather(t2v, [s4 + c1]) * w
        plsc.addupdate_scatter(acc2, [dq, dr + c0], m0)
        plsc.addupdate_scatter(acc2, [dq, dr + c1], m1)
        plsc.addupdate_scatter(acc2, [dq, dr + c2], w)

    cps = []
    for l in range(NTR):
        cps.append(pltpu.async_copy(acc2.at[pl.ds(128 * l, 128)],
                                    shacc.at[ibuf.at[l]], rsem, add=True))
    for cp in cps:
        cp.wait()
    plsc.subcore_barrier()
    pltpu.sync_copy(shacc.at[pl.ds(sid * ART, ART)],
                    p2_hbm.at[cid, pl.ds(sid * ART, ART)])


def _edge2(t2flat, src, dst):
    mesh = plsc.VectorSubcoreMesh(core_axis_name="c", subcore_axis_name="s")
    return pl.kernel(
        _edge2_body,
        out_type=jax.ShapeDtypeStruct((NC, AR, 16), jnp.float32),
        mesh=mesh,
        compiler_params=pltpu.CompilerParams(use_tc_tiling_on_sc=False,
                                             needs_layout_passes=False),
        scratch_types=[
            pltpu.VMEM_SHARED((AR, 16), jnp.float32),
            pltpu.VMEM((AR, 16), jnp.float32),
            pltpu.VMEM((N * 4,), jnp.float32),
            pltpu.VMEM((EPW,), jnp.int32),
            pltpu.VMEM((EPW,), jnp.int32),
            pltpu.VMEM((NTR, 128), jnp.int32),
            pltpu.SemaphoreType.DMA,
        ],
    )(t2flat, src, dst)


def _stage_e_body(p20_ref, p21_ref, t2_ref, b2_ref, out_ref):
    t2 = t2_ref[...]
    p2 = p20_ref[0] + p21_ref[0]
    wself = jnp.exp(_lrelu(t2[:, 2] + t2[:, 3]))
    s0 = p2[:, 0] + wself * t2[:, 0]
    s1 = p2[:, 1] + wself * t2[:, 1]
    den = p2[:, 2] + wself + 1e-16
    out_ref[...] = (jnp.stack([s0, s1], axis=1) / den[:, None]
                    + b2_ref[...][0])


def _stage_e(p2, t2, b2):
    return pl.pallas_call(
        _stage_e_body,
        grid=(GRID,),
        in_specs=[
            pl.BlockSpec((1, ROW_BLK, 4), lambda i: (0, i, 0)),
            pl.BlockSpec((1, ROW_BLK, 4), lambda i: (1, i, 0)),
            pl.BlockSpec((ROW_BLK, 4), lambda i: (i, 0)),
            pl.BlockSpec((1, OUT), lambda i: (0, 0)),
        ],
        out_specs=pl.BlockSpec((ROW_BLK, OUT), lambda i: (i, 0)),
        out_shape=jax.ShapeDtypeStruct((N, OUT), jnp.float32),
    )(p2, p2, t2, b2)


def kernel(x, edge_index, W1, att_src1, att_dst1, b1, W2, att_src2, att_dst2, b2):
    src = edge_index[0]
    dst = edge_index[1]
    eye = jnp.eye(HEADS, dtype=jnp.float32)
    a_s = (att_src1[:, :, None] * eye[:, None, :]).reshape(HF, HEADS)
    a_d = (att_dst1[:, :, None] * eye[:, None, :]).reshape(HF, HEADS)

    t1, d1 = _stage_a(x, W1, a_s, a_d)
    p1 = _edge1(t1, d1, src, dst)
    t2 = _stage_c(p1, t1, d1, b1.reshape(1, HF), W2, att_src2, att_dst2)
    p2 = _edge2(t2.reshape(N * 4), src, dst)
    p2c = p2.reshape(NC, AR * 16)[:, :N * 4].reshape(NC, N, 4)
    return _stage_e(p2c, t2, b2.reshape(1, OUT))
